# Initial kernel scaffold; baseline (speedup 1.0000x reference)
import jax
import jax.numpy as jnp
from jax.experimental import pallas as pl


def kernel(points, point_features, boxes3d):
    raise NotImplementedError("write your pallas kernel here")


# speedup vs baseline: 11.5555x; 11.5555x over previous
import functools

import jax
import jax.numpy as jnp
import numpy as np
from jax import lax
from jax.experimental import pallas as pl
from jax.experimental.pallas import tpu as pltpu
from jax.experimental.pallas import tpu_sc as plsc

_B, _N, _C, _M = 2, 16384, 128, 128
_NS = 512
_D = _C + 3
_NW = 32
_BOXES_PER_W = _B * _M // _NW
_IDXCAP = 544
_CHUNKS = _N // 16
_GROUP = 8
_NGROUP = _CHUNKS // _GROUP
_ROWWORDS = _NS * _D
_POOL_EXTRA_WIDTH = 1.0


def _sc_pool(pts_t, feats, params):
    mesh = plsc.VectorSubcoreMesh(core_axis_name="c", subcore_axis_name="s")

    @functools.partial(
        pl.kernel,
        mesh=mesh,
        out_type=[
            jax.ShapeDtypeStruct((_B * _M, _ROWWORDS), jnp.float32),
            jax.ShapeDtypeStruct((_B * _M,), jnp.int32),
        ],
        scratch_types=[
            pltpu.VMEM((3, _N), jnp.float32),
            pltpu.VMEM((_IDXCAP,), jnp.int32),
            pltpu.VMEM((4, 128), jnp.int32),
            pltpu.VMEM((2, 128, _C), jnp.float32),
            pltpu.VMEM((2, 128 * _D), jnp.float32),
            pltpu.VMEM((16,), jnp.int32),
            pltpu.VMEM((16,), jnp.int32),
            pltpu.VMEM((_BOXES_PER_W, 8, 16), jnp.float32),
            pltpu.SemaphoreType.DMA,
            pltpu.SemaphoreType.DMA,
            pltpu.SemaphoreType.DMA,
            pltpu.SemaphoreType.DMA,
        ],
        compiler_params=pltpu.CompilerParams(needs_layout_passes=False,
                                             use_tc_tiling_on_sc=False),
    )
    def k(pts_hbm, feat_hbm, par_hbm, out_hbm, flag_hbm,
          pts_v, idx_v, sel_v, feat_v, rows_v, flag_v, off_ref, par_v,
          semg0, semg1, semw0, semw1):
        c = lax.axis_index("c")
        s = lax.axis_index("s")
        wid = c * 16 + s
        b = c
        box0 = wid * _BOXES_PER_W

        pltpu.sync_copy(pts_hbm.at[b], pts_v)
        pltpu.sync_copy(par_hbm.at[pl.ds(box0, _BOXES_PER_W)], par_v)

        iota = lax.broadcasted_iota(jnp.int32, (16,), 0)
        row_base = b * _N

        def box_body(bi, flags):
            cxv = par_v[bi, 0, :]
            cyv = par_v[bi, 1, :]
            czv = par_v[bi, 2, :]
            dxv = par_v[bi, 3, :]
            dyv = par_v[bi, 4, :]
            dzv = par_v[bi, 5, :]
            cav = par_v[bi, 6, :]
            sav = par_v[bi, 7, :]

            off_ref[...] = jnp.zeros((16,), jnp.int32)

            def grp_body(g, carry):
                offv0 = off_ref[...]

                @pl.when(offv0[0] < _NS)
                def _scan_group():
                    offv = offv0
                    for u in range(_GROUP):
                        base = (g * _GROUP + u) * 16
                        xv = pts_v[0, pl.ds(base, 16)]
                        yv = pts_v[1, pl.ds(base, 16)]
                        zv = pts_v[2, pl.ds(base, 16)]
                        sx = xv - cxv
                        sy = yv - cyv
                        sz = zv - czv
                        xr = sx * cav - sy * sav
                        yr = sx * sav + sy * cav
                        m = ((jnp.abs(sz) <= dzv)
                             & (jnp.abs(xr) <= dxv)
                             & (jnp.abs(yr) <= dyv))
                        cs = plsc.cumsum(m.astype(jnp.int32))
                        pos = offv + cs - 1
                        wm = m & (pos < _IDXCAP)
                        plsc.store_scatter(idx_v, [pos], base + iota, mask=wm)
                        offv = offv + plsc.all_reduce_population_count(m)
                    off_ref[...] = offv

                return carry

            lax.fori_loop(0, _NGROUP, grp_body, 0)

            cnt_v = off_ref[...]
            cnt = cnt_v[0]
            safe_v = jnp.maximum(cnt_v, 1)
            zero16 = jnp.zeros((16,), jnp.int32)
            for j in range(_NS // 16):
                ar = j * 16 + iota
                selv = jnp.where(ar < cnt_v, ar, ar % safe_v)
                pidx = plsc.load_gather(idx_v, [selv])
                pidx = jnp.minimum(jnp.maximum(pidx, 0), _N - 1)
                sel_v[j // 8, pl.ds((j % 8) * 16, 16)] = row_base + pidx

            semg = (semg0, semg1)
            semw = (semw0, semw1)
            gathers = [None, None]
            writes = [None, None]
            gathers[0] = pltpu.async_copy(feat_hbm.at[sel_v.at[0]],
                                          feat_v.at[0], semg[0])
            for g4 in range(4):
                p = g4 % 2
                gathers[p].wait()
                if g4 + 1 < 4:
                    gathers[1 - p] = pltpu.async_copy(
                        feat_hbm.at[sel_v.at[g4 + 1]],
                        feat_v.at[1 - p], semg[1 - p])
                if writes[p] is not None:
                    writes[p].wait()
                def asm_body(r2, carry):
                    for h in range(2):
                        r = r2 * 2 + h
                        fbase = r * _D + 3
                        for kk in range(_C // 16):
                            rows_v[p, pl.ds(fbase + kk * 16, 16)] = (
                                feat_v[p, r, pl.ds(kk * 16, 16)])
                    return carry
                lax.fori_loop(0, 64, asm_body, 0)
                for kq in range(8):
                    selc = sel_v[g4, pl.ds(kq * 16, 16)]
                    pidx = selc - row_base
                    px = plsc.load_gather(pts_v, [zero16, pidx])
                    py = plsc.load_gather(pts_v, [zero16 + 1, pidx])
                    pz = plsc.load_gather(pts_v, [zero16 + 2, pidx])
                    rpos = (kq * 16 + iota) * _D
                    plsc.store_scatter(rows_v, [zero16 + p, rpos], px)
                    plsc.store_scatter(rows_v, [zero16 + p, rpos + 1], py)
                    plsc.store_scatter(rows_v, [zero16 + p, rpos + 2], pz)

                @pl.when(cnt == 0)
                def _zero_fill():
                    zf = jnp.zeros((16,), jnp.float32)
                    def zrow(i, carry):
                        rows_v[p, pl.ds(i * 16, 16)] = zf
                        return carry
                    lax.fori_loop(0, (128 * _D) // 16, zrow, 0)

                writes[p] = pltpu.async_copy(
                    rows_v.at[p],
                    out_hbm.at[box0 + bi, pl.ds(g4 * 128 * _D, 128 * _D)],
                    semw[p])
            for p in range(2):
                writes[p].wait()

            empty = jnp.full((16,), (cnt == 0).astype(jnp.int32), jnp.int32)
            flags = jnp.where(iota == bi, empty, flags)
            return flags

        flags = lax.fori_loop(0, _BOXES_PER_W, box_body,
                              jnp.zeros((16,), jnp.int32))
        flag_v[...] = flags
        pltpu.sync_copy(flag_v.at[pl.ds(0, _BOXES_PER_W)],
                        flag_hbm.at[pl.ds(box0, _BOXES_PER_W)])

    return k(pts_t, feats, params)


def kernel(points, point_features, boxes3d):
    B, N, _ = points.shape
    M = boxes3d.shape[1]

    eb = boxes3d.at[..., 3:6].add(_POOL_EXTRA_WIDTH)
    eb = eb.at[..., 2].add(-_POOL_EXTRA_WIDTH / 2.0)
    cx, cy, cz, dx, dy, dz, rz = [eb[..., i] for i in range(7)]
    czc = cz + dz / 2.0
    cosa = jnp.cos(-rz)
    sina = jnp.sin(-rz)
    params = jnp.stack([cx, cy, czc, dx / 2.0, dy / 2.0, dz / 2.0, cosa, sina],
                       axis=-1)
    params = params.reshape(B * M, 8, 1)
    params = jnp.broadcast_to(params, (B * M, 8, 16)).astype(jnp.float32)

    pts_t = jnp.transpose(points, (0, 2, 1))
    feats = point_features.reshape(B * N, _C)

    out, flags = _sc_pool(pts_t, feats, params)
    return out.reshape(B, M, _NS, _D), flags.reshape(B, M)
